# Initial kernel scaffold; baseline (speedup 1.0000x reference)
import jax
import jax.numpy as jnp
from jax.experimental import pallas as pl


def kernel(x, W, b):
    raise NotImplementedError("write your pallas kernel here")


# speedup vs baseline: 1.5291x; 1.5291x over previous
import math

import jax
import jax.numpy as jnp
from jax.experimental import pallas as pl

_N = 10000
_DEG = 32
_D = 128
_DELTA = 3.4965
_BN = 400


def _pna_kernel(x_ref, w_ref, b_ref, o_ref):
    deg = _DEG
    c1 = math.log(deg + 1) / _DELTA
    c2 = _DELTA / math.log(deg + 1)

    xb = x_ref[...]
    s = jnp.sum(xb, axis=1)
    sq = jnp.sum(xb * xb, axis=1)
    mx = jnp.max(xb, axis=1)
    mn = jnp.min(xb, axis=1)
    mean = s * (1.0 / deg)
    var = sq * (1.0 / deg) - mean * mean
    std = jnp.sqrt(jnp.maximum(var, 0.0))

    w = w_ref[...]
    w_eff = (
        w[0 : 4 * _D, :]
        + c1 * w[4 * _D : 8 * _D, :]
        + c2 * w[8 * _D : 12 * _D, :]
    )

    acc = jnp.dot(mean, w_eff[0 * _D : 1 * _D, :])
    acc += jnp.dot(mx, w_eff[1 * _D : 2 * _D, :])
    acc += jnp.dot(mn, w_eff[2 * _D : 3 * _D, :])
    acc += jnp.dot(std, w_eff[3 * _D : 4 * _D, :])
    o_ref[...] = acc + b_ref[...]


def kernel(x, W, b):
    n = x.shape[0]
    grid = (n // _BN,)
    b2 = b.reshape(1, _D)
    return pl.pallas_call(
        _pna_kernel,
        grid=grid,
        in_specs=[
            pl.BlockSpec((_BN, _DEG, _D), lambda i: (i, 0, 0)),
            pl.BlockSpec((12 * _D, _D), lambda i: (0, 0)),
            pl.BlockSpec((1, _D), lambda i: (0, 0)),
        ],
        out_specs=pl.BlockSpec((_BN, _D), lambda i: (i, 0)),
        out_shape=jax.ShapeDtypeStruct((n, _D), jnp.float32),
    )(x, W, b2)


# speedup vs baseline: 2.1231x; 1.3885x over previous
import math

import jax
import jax.numpy as jnp
from jax.experimental import pallas as pl

_N = 10000
_DEG = 32
_D = 128
_DELTA = 3.4965
_BN = 1000


def _pna_kernel(x_ref, w_ref, b_ref, o_ref):
    deg = _DEG
    c1 = math.log(deg + 1) / _DELTA
    c2 = _DELTA / math.log(deg + 1)

    xb = x_ref[...]
    s = jnp.sum(xb, axis=1)
    sq = jnp.sum(xb * xb, axis=1)
    mx = jnp.max(xb, axis=1)
    mn = jnp.min(xb, axis=1)
    mean = s * (1.0 / deg)
    var = sq * (1.0 / deg) - mean * mean
    std = jnp.sqrt(jnp.maximum(var, 0.0))

    w = w_ref[...]
    w_eff = (
        w[0 : 4 * _D, :]
        + c1 * w[4 * _D : 8 * _D, :]
        + c2 * w[8 * _D : 12 * _D, :]
    )

    acc = jnp.dot(mean, w_eff[0 * _D : 1 * _D, :])
    acc += jnp.dot(mx, w_eff[1 * _D : 2 * _D, :])
    acc += jnp.dot(mn, w_eff[2 * _D : 3 * _D, :])
    acc += jnp.dot(std, w_eff[3 * _D : 4 * _D, :])
    o_ref[...] = acc + b_ref[...]


def kernel(x, W, b):
    n = x.shape[0]
    grid = (n // _BN,)
    b2 = b.reshape(1, _D)
    return pl.pallas_call(
        _pna_kernel,
        grid=grid,
        in_specs=[
            pl.BlockSpec((_BN, _DEG, _D), lambda i: (i, 0, 0)),
            pl.BlockSpec((12 * _D, _D), lambda i: (0, 0)),
            pl.BlockSpec((1, _D), lambda i: (0, 0)),
        ],
        out_specs=pl.BlockSpec((_BN, _D), lambda i: (i, 0)),
        out_shape=jax.ShapeDtypeStruct((n, _D), jnp.float32),
    )(x, W, b2)
